# Initial kernel scaffold; baseline (speedup 1.0000x reference)
import jax
import jax.numpy as jnp
from jax.experimental import pallas as pl


def kernel(x, abs_x, idx, Wq, Wk, Wv, Wq_nl, Wk_nl, Wv_nl):
    raise NotImplementedError("write your pallas kernel here")


# speedup vs baseline: 1.1921x; 1.1921x over previous
import jax
import jax.numpy as jnp
from jax import lax
from jax.experimental import pallas as pl
from jax.experimental.pallas import tpu as pltpu
from jax.experimental.pallas import tpu_sc as plsc

_B, _C, _N, _K = 2, 256, 2048, 16
_G = 4
_LC, _NLC = 192, 64
_GC = _LC // _G
_NGC = _NLC // _G
_TN = 128
_F32 = jnp.float32
_HI = lax.Precision.HIGHEST

_NC, _NS = 2, 16
_JOBS = _B * _G
_PARTS = _NC * _NS // _JOBS
_SEG = _N * _K // _PARTS


def _local_body(x_ref, wq_ref, wk_ref, wv_ref, pout_ref, outl_ref):
    xb = x_ref[0]
    q = jnp.dot(wq_ref[...], xb, preferred_element_type=_F32)
    k = jnp.dot(wk_ref[...], xb, preferred_element_type=_F32)
    v = jnp.dot(wv_ref[...], xb, preferred_element_type=_F32)
    s = q * k
    grp = (lax.broadcasted_iota(jnp.int32, (_G, _LC), 1) // _GC
           == lax.broadcasted_iota(jnp.int32, (_G, _LC), 0)).astype(_F32)
    logits = lax.dot(grp, s, precision=_HI)
    l3 = logits.reshape(_G, _TN, _K)
    m = jnp.max(l3, axis=-1, keepdims=True)
    e = jnp.exp(l3 - m)
    p3 = e / jnp.sum(e, axis=-1, keepdims=True)
    pf = lax.dot(grp.T, p3.reshape(_G, _TN * _K), precision=_HI)
    seg = (lax.broadcasted_iota(jnp.int32, (_TN * _K, _TN), 0) // _K
           == lax.broadcasted_iota(jnp.int32, (_TN * _K, _TN), 1)).astype(_F32)
    outl_ref[0] = lax.dot(pf * v, seg, precision=_HI)
    pout_ref[0] = p3


_ROWS_PER_CHUNK = 32


def _sc_body(pout_hbm, idx_hbm, msk_hbm, vals_hbm, sidx_hbm, score_hbm,
             idx_v, val_v, msk_v, acc_v, tmp_v, o16f, o16i, score_v,
             shared):
    c = lax.axis_index("c")
    s = lax.axis_index("s")
    job = c * (_JOBS // _NC) + s // _PARTS
    part = s % _PARTS
    b = job // _G
    base = part * _SEG
    pltpu.sync_copy(pout_hbm.at[job, pl.ds(base, _SEG)], val_v)
    pltpu.sync_copy(idx_hbm.at[b, pl.ds(base, _SEG)], idx_v)
    pltpu.sync_copy(msk_hbm.at[job, pl.ds(base, _SEG)], msk_v)

    def zero_body(i, carry):
        acc_v[pl.ds(i * 16, 16)] = jnp.zeros((16,), _F32)
        return carry

    lax.fori_loop(0, _N // 16, zero_body, 0)

    def row_body(r, carry):
        o = r * _K
        iv = idx_v[pl.ds(o, _K)]
        ov = val_v[pl.ds(o, _K)]
        mv = msk_v[pl.ds(o, _K)] != 0
        plsc.addupdate_scatter(acc_v, [iv], ov, mask=mv)
        return carry

    lax.fori_loop(0, _SEG // _K, row_body, 0)

    pltpu.sync_copy(acc_v, shared.at[s])
    plsc.subcore_barrier()

    @pl.when(part == 0)
    def _():
        for i in range(_PARTS):
            pltpu.sync_copy(shared.at[s + i], tmp_v.at[i])

        def tk_body(i, carry):
            tv, ti = carry
            o = i * 16
            v = (tmp_v[0, pl.ds(o, 16)] + tmp_v[1, pl.ds(o, 16)]
                 + tmp_v[2, pl.ds(o, 16)] + tmp_v[3, pl.ds(o, 16)])
            score_v[pl.ds(o, 16)] = v
            ids = lax.iota(jnp.int32, 16) + o
            mx = jnp.max(v)
            mn = jnp.min(tv)

            def merge(args):
                tv, ti, v, ids = args
                vs, is_ = plsc.sort_key_val(v, ids)
                take_a = tv >= vs
                mk = jnp.where(take_a, tv, vs)
                mi = jnp.where(take_a, ti, is_)
                nk, ni = plsc.sort_key_val(mk, mi, descending=True)
                return nk, ni

            def keep(args):
                tv, ti, _, _ = args
                return tv, ti

            return lax.cond(mx > mn, merge, keep, (tv, ti, v, ids))

        tv0 = jnp.full((16,), -1.0, _F32)
        ti0 = jnp.zeros((16,), jnp.int32)
        tv, ti = lax.fori_loop(0, _N // 16, tk_body, (tv0, ti0))
        o16f[...] = tv
        o16i[...] = ti
        pltpu.sync_copy(o16f, vals_hbm.at[job])
        pltpu.sync_copy(o16i, sidx_hbm.at[job])
        pltpu.sync_copy(score_v, score_hbm.at[job])


def _nl_body(absx_ref, wq_ref, wk_ref, wv_ref, sidx_ref, vals_ref, out_ref):
    ax = absx_ref[0]
    sel = sidx_ref[0, 0, 0]
    sval = vals_ref[0, 0, 0]
    oh = (lax.broadcasted_iota(jnp.int32, (_N, _K), 0) == sel[None, :]).astype(_F32)
    ag = lax.dot(ax, oh, precision=_HI)
    q = jnp.dot(wq_ref[...], ax, preferred_element_type=_F32)
    kg = jnp.dot(wk_ref[...], ag, preferred_element_type=_F32)
    vg = jnp.dot(wv_ref[...], ag, preferred_element_type=_F32)
    vg = vg * jnp.tanh(sval)[None, :]
    attT = lax.dot_general(kg, q, (((0,), (0,)), ((), ())),
                           preferred_element_type=_F32)
    m = jnp.max(attT, axis=0, keepdims=True)
    e = jnp.exp(attT - m)
    p = e / jnp.sum(e, axis=0, keepdims=True)
    out_ref[0, 0] = jnp.dot(vg, p, preferred_element_type=_F32)


_SC_MESH = plsc.VectorSubcoreMesh(core_axis_name="c", subcore_axis_name="s",
                                  num_cores=_NC, num_subcores=_NS)


def _score_topk(pout2, idx2, keep2):
    return pl.kernel(
        _sc_body,
        out_type=[
            jax.ShapeDtypeStruct((_JOBS, _K), _F32),
            jax.ShapeDtypeStruct((_JOBS, _K), jnp.int32),
            jax.ShapeDtypeStruct((_JOBS, _N), _F32),
        ],
        mesh=_SC_MESH,
        compiler_params=pltpu.CompilerParams(needs_layout_passes=False),
        scratch_types=[
            pltpu.VMEM((_SEG,), jnp.int32),
            pltpu.VMEM((_SEG,), _F32),
            pltpu.VMEM((_SEG,), jnp.int32),
            pltpu.VMEM((_N,), _F32),
            pltpu.VMEM((_PARTS, _N), _F32),
            pltpu.VMEM((_K,), _F32),
            pltpu.VMEM((_K,), jnp.int32),
            pltpu.VMEM((_N,), _F32),
            pltpu.VMEM_SHARED((_NS, _N), _F32),
        ],
    )(pout2, idx2, keep2)


def _stage_local(x, Wq, Wk, Wv):
    x2 = x.reshape(_B, _C, _N * _K)
    return pl.pallas_call(
        _local_body,
        grid=(_B, _N // _TN),
        in_specs=[
            pl.BlockSpec((1, _C, _TN * _K), lambda b, j: (b, 0, j)),
            pl.BlockSpec((_LC, _C), lambda b, j: (0, 0)),
            pl.BlockSpec((_LC, _C), lambda b, j: (0, 0)),
            pl.BlockSpec((_LC, _C), lambda b, j: (0, 0)),
        ],
        out_specs=[
            pl.BlockSpec((1, _G, _TN, _K), lambda b, j: (b, 0, j, 0)),
            pl.BlockSpec((1, _LC, _TN), lambda b, j: (b, 0, j)),
        ],
        out_shape=[
            jax.ShapeDtypeStruct((_B, _G, _N, _K), _F32),
            jax.ShapeDtypeStruct((_B, _LC, _N), _F32),
        ],
    )(x2, Wq, Wk, Wv)


def _keep_mask(idx3):
    bi = jnp.arange(_B, dtype=jnp.int32)[:, None, None, None]
    gi = jnp.arange(_G, dtype=jnp.int32)[None, :, None, None]
    ni = jnp.arange(_N, dtype=jnp.int32)[None, None, :, None]
    key4 = ((bi * _G + gi) * _N + ni) * _N + idx3[:, None]
    keyf = key4.reshape(-1)
    pos = jnp.arange(keyf.shape[0], dtype=jnp.int32)
    sk, sp = lax.sort((keyf, pos), num_keys=1, dimension=0, is_stable=False)
    is_last = jnp.concatenate(
        [sk[1:] != sk[:-1], jnp.ones((1,), jnp.bool_)]).astype(jnp.int32)
    keep = jnp.zeros_like(pos).at[sp].set(is_last, unique_indices=True,
                                          mode="promise_in_bounds")
    return keep.reshape(_JOBS, _N * _K)


def kernel(x, abs_x, idx, Wq, Wk, Wv, Wq_nl, Wk_nl, Wv_nl):
    idx3 = idx.reshape(_B, _N, _K)
    pout, out_l = _stage_local(x, Wq, Wk, Wv)
    pout2 = pout.reshape(_JOBS, _N * _K)
    idx2 = idx3.reshape(_B, _N * _K)
    keep2 = _keep_mask(idx3)
    vals, sidx, _score = _score_topk(pout2, idx2, keep2)

    s4 = sidx.reshape(_B, _G, 1, _K)
    v4 = vals.reshape(_B, _G, 1, _K)
    ax3 = abs_x.reshape(_B, _C // 2, _N)
    out_all = pl.pallas_call(
        _nl_body,
        grid=(_B, _G),
        in_specs=[
            pl.BlockSpec((1, _C // 2, _N), lambda b, g: (b, 0, 0)),
            pl.BlockSpec((_NGC, _C // 2), lambda b, g: (g, 0)),
            pl.BlockSpec((_NGC, _C // 2), lambda b, g: (g, 0)),
            pl.BlockSpec((_NGC, _C // 2), lambda b, g: (g, 0)),
            pl.BlockSpec((1, 1, 1, _K), lambda b, g: (b, g, 0, 0)),
            pl.BlockSpec((1, 1, 1, _K), lambda b, g: (b, g, 0, 0)),
        ],
        out_specs=pl.BlockSpec((1, 1, _NGC, _N), lambda b, g: (b, g, 0, 0)),
        out_shape=jax.ShapeDtypeStruct((_B, _G, _NGC, _N), _F32),
    )(ax3, Wq_nl, Wk_nl, Wv_nl, s4, v4)

    return jnp.concatenate(
        [out_l.reshape(_B, _LC, _N, 1), out_all.reshape(_B, _NLC, _N, 1)],
        axis=1)


# speedup vs baseline: 3.5942x; 3.0150x over previous
import jax
import jax.numpy as jnp
from jax import lax
from jax.experimental import pallas as pl
from jax.experimental.pallas import tpu as pltpu
from jax.experimental.pallas import tpu_sc as plsc

_B, _C, _N, _K = 2, 256, 2048, 16
_G = 4
_LC, _NLC = 192, 64
_GC = _LC // _G
_NGC = _NLC // _G
_TN = 128
_F32 = jnp.float32
_HI = lax.Precision.HIGHEST

_NC, _NS = 2, 16
_JOBS = _B * _G
_PARTS = _NC * _NS // _JOBS
_SEG = _N * _K // _PARTS


def _local_body(x_ref, wq_ref, wk_ref, wv_ref, pout_ref, outl_ref):
    xb = x_ref[0]
    q = jnp.dot(wq_ref[...], xb, preferred_element_type=_F32)
    k = jnp.dot(wk_ref[...], xb, preferred_element_type=_F32)
    v = jnp.dot(wv_ref[...], xb, preferred_element_type=_F32)
    s = q * k
    grp = (lax.broadcasted_iota(jnp.int32, (_G, _LC), 1) // _GC
           == lax.broadcasted_iota(jnp.int32, (_G, _LC), 0)).astype(_F32)
    logits = lax.dot(grp, s, precision=_HI)
    l3 = logits.reshape(_G, _TN, _K)
    m = jnp.max(l3, axis=-1, keepdims=True)
    e = jnp.exp(l3 - m)
    p3 = e / jnp.sum(e, axis=-1, keepdims=True)
    pf = lax.dot(grp.T, p3.reshape(_G, _TN * _K), precision=_HI)
    seg = (lax.broadcasted_iota(jnp.int32, (_TN * _K, _TN), 0) // _K
           == lax.broadcasted_iota(jnp.int32, (_TN * _K, _TN), 1)).astype(_F32)
    outl_ref[0] = lax.dot(pf * v, seg, precision=_HI)
    pout_ref[0] = p3


_ROWS_PER_CHUNK = 32


def _sc_body(pout_hbm, idx_hbm, sp_hbm, il_hbm, vals_hbm, sidx_hbm, score_hbm,
             idx_v, val_v, sp_v, il_v, msk_v, acc_v, tmp_v, o16f, o16i,
             score_v, shared):
    c = lax.axis_index("c")
    s = lax.axis_index("s")
    job = c * (_JOBS // _NC) + s // _PARTS
    part = s % _PARTS
    b = job // _G
    base = part * _SEG
    pltpu.sync_copy(pout_hbm.at[job, pl.ds(base, _SEG)], val_v)
    pltpu.sync_copy(idx_hbm.at[b, pl.ds(base, _SEG)], idx_v)
    pltpu.sync_copy(sp_hbm.at[job, pl.ds(base, _SEG)], sp_v)
    pltpu.sync_copy(il_hbm.at[job, pl.ds(base, _SEG)], il_v)

    def zero_body(i, carry):
        acc_v[pl.ds(i * 16, 16)] = jnp.zeros((16,), _F32)
        return carry

    lax.fori_loop(0, _N // 16, zero_body, 0)

    pbase = job * (_N * _K) + base

    def inv_body(r, carry):
        o = r * _K
        pv = sp_v[pl.ds(o, _K)] - pbase
        fv = il_v[pl.ds(o, _K)]
        plsc.store_scatter(msk_v, [pv], fv)
        return carry

    lax.fori_loop(0, _SEG // _K, inv_body, 0)

    def row_body(r, carry):
        o = r * _K
        iv = idx_v[pl.ds(o, _K)]
        ov = val_v[pl.ds(o, _K)]
        mv = msk_v[pl.ds(o, _K)] != 0
        plsc.addupdate_scatter(acc_v, [iv], ov, mask=mv)
        return carry

    lax.fori_loop(0, _SEG // _K, row_body, 0)

    pltpu.sync_copy(acc_v, shared.at[s])
    plsc.subcore_barrier()

    @pl.when(part == 0)
    def _():
        for i in range(_PARTS):
            pltpu.sync_copy(shared.at[s + i], tmp_v.at[i])

        def tk_body(i, carry):
            tv, ti = carry
            o = i * 16
            v = (tmp_v[0, pl.ds(o, 16)] + tmp_v[1, pl.ds(o, 16)]
                 + tmp_v[2, pl.ds(o, 16)] + tmp_v[3, pl.ds(o, 16)])
            score_v[pl.ds(o, 16)] = v
            ids = lax.iota(jnp.int32, 16) + o
            mx = jnp.max(v)
            mn = jnp.min(tv)

            def merge(args):
                tv, ti, v, ids = args
                vs, is_ = plsc.sort_key_val(v, ids)
                take_a = tv >= vs
                mk = jnp.where(take_a, tv, vs)
                mi = jnp.where(take_a, ti, is_)
                nk, ni = plsc.sort_key_val(mk, mi, descending=True)
                return nk, ni

            def keep(args):
                tv, ti, _, _ = args
                return tv, ti

            return lax.cond(mx > mn, merge, keep, (tv, ti, v, ids))

        tv0 = jnp.full((16,), -1.0, _F32)
        ti0 = jnp.zeros((16,), jnp.int32)
        tv, ti = lax.fori_loop(0, _N // 16, tk_body, (tv0, ti0))
        o16f[...] = tv
        o16i[...] = ti
        pltpu.sync_copy(o16f, vals_hbm.at[job])
        pltpu.sync_copy(o16i, sidx_hbm.at[job])
        pltpu.sync_copy(score_v, score_hbm.at[job])


def _nl_body(absx_ref, wq_ref, wk_ref, wv_ref, sidx_ref, vals_ref, out_ref):
    ax = absx_ref[0]
    sel = sidx_ref[0, 0, 0]
    sval = vals_ref[0, 0, 0]
    oh = (lax.broadcasted_iota(jnp.int32, (_N, _K), 0) == sel[None, :]).astype(_F32)
    ag = lax.dot(ax, oh, precision=_HI)
    q = jnp.dot(wq_ref[...], ax, preferred_element_type=_F32)
    kg = jnp.dot(wk_ref[...], ag, preferred_element_type=_F32)
    vg = jnp.dot(wv_ref[...], ag, preferred_element_type=_F32)
    vg = vg * jnp.tanh(sval)[None, :]
    attT = lax.dot_general(kg, q, (((0,), (0,)), ((), ())),
                           preferred_element_type=_F32)
    m = jnp.max(attT, axis=0, keepdims=True)
    e = jnp.exp(attT - m)
    p = e / jnp.sum(e, axis=0, keepdims=True)
    out_ref[0, 0] = jnp.dot(vg, p, preferred_element_type=_F32)


_SC_MESH = plsc.VectorSubcoreMesh(core_axis_name="c", subcore_axis_name="s",
                                  num_cores=_NC, num_subcores=_NS)


def _score_topk(pout2, idx2, sp2, il2):
    return pl.kernel(
        _sc_body,
        out_type=[
            jax.ShapeDtypeStruct((_JOBS, _K), _F32),
            jax.ShapeDtypeStruct((_JOBS, _K), jnp.int32),
            jax.ShapeDtypeStruct((_JOBS, _N), _F32),
        ],
        mesh=_SC_MESH,
        compiler_params=pltpu.CompilerParams(needs_layout_passes=False),
        scratch_types=[
            pltpu.VMEM((_SEG,), jnp.int32),
            pltpu.VMEM((_SEG,), _F32),
            pltpu.VMEM((_SEG,), jnp.int32),
            pltpu.VMEM((_SEG,), jnp.int32),
            pltpu.VMEM((_SEG,), jnp.int32),
            pltpu.VMEM((_N,), _F32),
            pltpu.VMEM((_PARTS, _N), _F32),
            pltpu.VMEM((_K,), _F32),
            pltpu.VMEM((_K,), jnp.int32),
            pltpu.VMEM((_N,), _F32),
            pltpu.VMEM_SHARED((_NS, _N), _F32),
        ],
    )(pout2, idx2, sp2, il2)


def _stage_local(x, Wq, Wk, Wv):
    x2 = x.reshape(_B, _C, _N * _K)
    return pl.pallas_call(
        _local_body,
        grid=(_B, _N // _TN),
        in_specs=[
            pl.BlockSpec((1, _C, _TN * _K), lambda b, j: (b, 0, j)),
            pl.BlockSpec((_LC, _C), lambda b, j: (0, 0)),
            pl.BlockSpec((_LC, _C), lambda b, j: (0, 0)),
            pl.BlockSpec((_LC, _C), lambda b, j: (0, 0)),
        ],
        out_specs=[
            pl.BlockSpec((1, _G, _TN, _K), lambda b, j: (b, 0, j, 0)),
            pl.BlockSpec((1, _LC, _TN), lambda b, j: (b, 0, j)),
        ],
        out_shape=[
            jax.ShapeDtypeStruct((_B, _G, _N, _K), _F32),
            jax.ShapeDtypeStruct((_B, _LC, _N), _F32),
        ],
    )(x2, Wq, Wk, Wv)


def _keep_mask_sorted(idx3):
    bi = jnp.arange(_B, dtype=jnp.int32)[:, None, None, None]
    gi = jnp.arange(_G, dtype=jnp.int32)[None, :, None, None]
    ni = jnp.arange(_N, dtype=jnp.int32)[None, None, :, None]
    key4 = ((bi * _G + gi) * _N + ni) * _N + idx3[:, None]
    keyf = key4.reshape(-1)
    pos = jnp.arange(keyf.shape[0], dtype=jnp.int32)
    sk, sp = lax.sort((keyf, pos), num_keys=1, dimension=0, is_stable=False)
    is_last = jnp.concatenate(
        [sk[1:] != sk[:-1], jnp.ones((1,), jnp.bool_)]).astype(jnp.int32)
    return sp.reshape(_JOBS, _N * _K), is_last.reshape(_JOBS, _N * _K)


def kernel(x, abs_x, idx, Wq, Wk, Wv, Wq_nl, Wk_nl, Wv_nl):
    idx3 = idx.reshape(_B, _N, _K)
    pout, out_l = _stage_local(x, Wq, Wk, Wv)
    pout2 = pout.reshape(_JOBS, _N * _K)
    idx2 = idx3.reshape(_B, _N * _K)
    sp2, il2 = _keep_mask_sorted(idx3)
    vals, sidx, _score = _score_topk(pout2, idx2, sp2, il2)

    s4 = sidx.reshape(_B, _G, 1, _K)
    v4 = vals.reshape(_B, _G, 1, _K)
    ax3 = abs_x.reshape(_B, _C // 2, _N)
    out_all = pl.pallas_call(
        _nl_body,
        grid=(_B, _G),
        in_specs=[
            pl.BlockSpec((1, _C // 2, _N), lambda b, g: (b, 0, 0)),
            pl.BlockSpec((_NGC, _C // 2), lambda b, g: (g, 0)),
            pl.BlockSpec((_NGC, _C // 2), lambda b, g: (g, 0)),
            pl.BlockSpec((_NGC, _C // 2), lambda b, g: (g, 0)),
            pl.BlockSpec((1, 1, 1, _K), lambda b, g: (b, g, 0, 0)),
            pl.BlockSpec((1, 1, 1, _K), lambda b, g: (b, g, 0, 0)),
        ],
        out_specs=pl.BlockSpec((1, 1, _NGC, _N), lambda b, g: (b, g, 0, 0)),
        out_shape=jax.ShapeDtypeStruct((_B, _G, _NGC, _N), _F32),
    )(ax3, Wq_nl, Wk_nl, Wv_nl, s4, v4)

    return jnp.concatenate(
        [out_l.reshape(_B, _LC, _N, 1), out_all.reshape(_B, _NLC, _N, 1)],
        axis=1)
